# baseline (device time: 73916 ns/iter reference)
import jax
import jax.numpy as jnp
from jax import lax
from jax.experimental import pallas as pl
from jax.experimental.pallas import tpu as pltpu


def kernel(Q, K, V, bt, lens):
    B, _, H, D = Q.shape
    P_loc, BS = K.shape[0], K.shape[1]
    NB = bt.shape[1]
    KT = P_loc * BS
    HD = H * D
    HB = H * B
    scale = D ** -0.5

    Qt = Q.reshape(B, H, D).transpose(1, 0, 2) * scale
    Qbig = jnp.einsum("hbd,hg->hbgd", Qt, jnp.eye(H, dtype=Q.dtype))
    Qbig = Qbig.reshape(HB, HD)
    K3 = K.reshape(KT, H, D)
    V3 = V.reshape(KT, H, D)
    lens2 = lens.reshape(B, 1)

    def body(q_ref, k_ref, v_ref, bt_ref, lens_ref, out_ref,
             k2_ref, v2_ref, send_buf, recv_buf, send_sem, recv_sem):
        my_x = lax.axis_index("x")
        my_y = lax.axis_index("y")
        peer = (my_x, 1 - my_y)

        barrier = pltpu.get_barrier_semaphore()
        pl.semaphore_signal(barrier, inc=1, device_id=peer,
                            device_id_type=pl.DeviceIdType.MESH)
        pl.semaphore_wait(barrier, 1)

        for h in range(H):
            sl = pl.ds(h * D, D)
            k2_ref[:, sl] = k_ref[:, h, :]
            v2_ref[:, sl] = v_ref[:, h, :]

        valid = lax.broadcasted_iota(jnp.int32, (B, NB), 1) < lens_ref[:, :]
        pg = my_y * P_loc + lax.broadcasted_iota(jnp.int32, (P_loc, B, NB), 0)
        hit = (bt_ref[:, :][None, :, :] == pg) & valid[None, :, :]
        counts = jnp.sum(jnp.where(hit, 1.0, 0.0), axis=2)

        kp = lax.broadcasted_iota(jnp.int32, (P_loc, KT), 1) // BS
        pp = lax.broadcasted_iota(jnp.int32, (P_loc, KT), 0)
        E = jnp.where(kp == pp, 1.0, 0.0)
        w = lax.dot_general(counts, E, (((0,), (0,)), ((), ())),
                            preferred_element_type=jnp.float32)
        rb = lax.broadcasted_iota(jnp.int32, (B, HB), 1) % B
        bb = lax.broadcasted_iota(jnp.int32, (B, HB), 0)
        T = jnp.where(rb == bb, 1.0, 0.0)
        wbig = lax.dot_general(T, w, (((0,), (0,)), ((), ())),
                               preferred_element_type=jnp.float32)

        s = lax.dot_general(q_ref[:, :], k2_ref[:, :],
                            (((1,), (1,)), ((), ())),
                            preferred_element_type=jnp.float32)
        p = jnp.exp(s) * wbig

        obig = lax.dot_general(p, v2_ref[:, :], (((1,), (0,)), ((), ())),
                               preferred_element_type=jnp.float32)
        lbig = lax.dot_general(p, jnp.ones((KT, D), jnp.float32),
                               (((1,), (0,)), ((), ())),
                               preferred_element_type=jnp.float32)

        for h in range(H):
            sl = pl.ds(h * D, D)
            send_buf[:, sl] = obig[h * B:(h + 1) * B, h * D:(h + 1) * D]
            send_buf[:, pl.ds(HD + h * D, D)] = lbig[h * B:(h + 1) * B, :]

        rdma = pltpu.make_async_remote_copy(
            src_ref=send_buf, dst_ref=recv_buf,
            send_sem=send_sem, recv_sem=recv_sem,
            device_id=peer, device_id_type=pl.DeviceIdType.MESH)
        rdma.start()
        rdma.wait()

        o_tot = send_buf[:, :HD] + recv_buf[:, :HD]
        l_tot = send_buf[:, HD:] + recv_buf[:, HD:]
        out_ref[:, :] = o_tot / l_tot

    out = pl.pallas_call(
        body,
        out_shape=jax.ShapeDtypeStruct((B, HD), jnp.float32),
        in_specs=[pl.BlockSpec(memory_space=pltpu.VMEM)] * 5,
        out_specs=pl.BlockSpec(memory_space=pltpu.VMEM),
        scratch_shapes=[
            pltpu.VMEM((KT, HD), jnp.float32),
            pltpu.VMEM((KT, HD), jnp.float32),
            pltpu.VMEM((B, 2 * HD), jnp.float32),
            pltpu.VMEM((B, 2 * HD), jnp.float32),
            pltpu.SemaphoreType.DMA,
            pltpu.SemaphoreType.DMA,
        ],
        compiler_params=pltpu.CompilerParams(
            collective_id=0, vmem_limit_bytes=100 * 1024 * 1024),
    )(Qbig, K3, V3, bt, lens2)

    return out.reshape(B, 1, H, D)


# device time: 72087 ns/iter; 1.0254x vs baseline; 1.0254x over previous
import jax
import jax.numpy as jnp
from jax import lax
from jax.experimental import pallas as pl
from jax.experimental.pallas import tpu as pltpu


def kernel(Q, K, V, bt, lens):
    B, _, H, D = Q.shape
    P_loc, BS = K.shape[0], K.shape[1]
    NB = bt.shape[1]
    KT = P_loc * BS
    HD = H * D
    Hh = H // 2
    HDh = Hh * D
    HBh = Hh * B
    scale = D ** -0.5

    mx = lax.axis_index("x")

    Qt = Q.reshape(B, H, D).transpose(1, 0, 2) * scale
    Qh = lax.dynamic_slice_in_dim(Qt, Hh * mx, Hh, axis=0)
    Qbig = jnp.einsum("hbd,hg->hbgd", Qh, jnp.eye(Hh, dtype=Q.dtype))
    Qbig = Qbig.reshape(HBh, HDh)

    K3 = K.reshape(KT, H, D)
    V3 = V.reshape(KT, H, D)
    K2 = lax.dynamic_slice_in_dim(K3, Hh * mx, Hh, axis=1).reshape(KT, HDh)
    V2 = lax.dynamic_slice_in_dim(V3, Hh * mx, Hh, axis=1).reshape(KT, HDh)
    lens2 = lens.reshape(B, 1)

    def body(q_ref, k_ref, v_ref, bt_ref, lens_ref, out_ref,
             sendy, recvy, sendx, recvx, sems_s, sems_r):
        my_x = lax.axis_index("x")
        my_y = lax.axis_index("y")
        ypeer = (my_x, 1 - my_y)
        xpeer = (1 - my_x, my_y)

        barrier = pltpu.get_barrier_semaphore()
        for nbr in (ypeer, xpeer):
            pl.semaphore_signal(barrier, inc=1, device_id=nbr,
                                device_id_type=pl.DeviceIdType.MESH)
        pl.semaphore_wait(barrier, 2)

        valid = lax.broadcasted_iota(jnp.int32, (B, NB), 1) < lens_ref[:, :]
        pg = my_y * P_loc + lax.broadcasted_iota(jnp.int32, (P_loc, B, NB), 0)
        hit = (bt_ref[:, :][None, :, :] == pg) & valid[None, :, :]
        counts = jnp.sum(jnp.where(hit, 1.0, 0.0), axis=2)

        kp = lax.broadcasted_iota(jnp.int32, (P_loc, KT), 1) // BS
        pp = lax.broadcasted_iota(jnp.int32, (P_loc, KT), 0)
        E = jnp.where(kp == pp, 1.0, 0.0)
        w = lax.dot_general(counts, E, (((0,), (0,)), ((), ())),
                            preferred_element_type=jnp.float32)
        rb = lax.broadcasted_iota(jnp.int32, (B, HBh), 1) % B
        bb = lax.broadcasted_iota(jnp.int32, (B, HBh), 0)
        T = jnp.where(rb == bb, 1.0, 0.0)
        wbig = lax.dot_general(T, w, (((0,), (0,)), ((), ())),
                               preferred_element_type=jnp.float32)

        s = lax.dot_general(q_ref[:, :], k_ref[:, :],
                            (((1,), (1,)), ((), ())),
                            preferred_element_type=jnp.float32)
        p = jnp.exp(s) * wbig

        obig = lax.dot_general(p, v_ref[:, :], (((1,), (0,)), ((), ())),
                               preferred_element_type=jnp.float32)
        lbig = lax.dot_general(p, jnp.ones((KT, D), jnp.float32),
                               (((1,), (0,)), ((), ())),
                               preferred_element_type=jnp.float32)

        for hl in range(Hh):
            sl = pl.ds(hl * D, D)
            sendy[:, sl] = obig[hl * B:(hl + 1) * B, hl * D:(hl + 1) * D]
            sendy[:, pl.ds(HDh + hl * D, D)] = lbig[hl * B:(hl + 1) * B, :]

        rdma_y = pltpu.make_async_remote_copy(
            src_ref=sendy, dst_ref=recvy,
            send_sem=sems_s.at[0], recv_sem=sems_r.at[0],
            device_id=ypeer, device_id_type=pl.DeviceIdType.MESH)
        rdma_y.start()
        rdma_y.wait()

        o_tot = sendy[:, :HDh] + recvy[:, :HDh]
        l_tot = sendy[:, HDh:] + recvy[:, HDh:]
        sendx[:, :] = o_tot / l_tot

        rdma_x = pltpu.make_async_remote_copy(
            src_ref=sendx, dst_ref=recvx,
            send_sem=sems_s.at[1], recv_sem=sems_r.at[1],
            device_id=xpeer, device_id_type=pl.DeviceIdType.MESH)
        rdma_x.start()
        rdma_x.wait()

        @pl.when(my_x == 0)
        def _():
            out_ref[:, :HDh] = sendx[:, :]
            out_ref[:, HDh:] = recvx[:, :]

        @pl.when(my_x == 1)
        def _():
            out_ref[:, :HDh] = recvx[:, :]
            out_ref[:, HDh:] = sendx[:, :]

    out = pl.pallas_call(
        body,
        out_shape=jax.ShapeDtypeStruct((B, HD), jnp.float32),
        in_specs=[pl.BlockSpec(memory_space=pltpu.VMEM)] * 5,
        out_specs=pl.BlockSpec(memory_space=pltpu.VMEM),
        scratch_shapes=[
            pltpu.VMEM((B, 2 * HDh), jnp.float32),
            pltpu.VMEM((B, 2 * HDh), jnp.float32),
            pltpu.VMEM((B, HDh), jnp.float32),
            pltpu.VMEM((B, HDh), jnp.float32),
            pltpu.SemaphoreType.DMA((2,)),
            pltpu.SemaphoreType.DMA((2,)),
        ],
        compiler_params=pltpu.CompilerParams(
            collective_id=0, vmem_limit_bytes=100 * 1024 * 1024),
    )(Qbig, K2, V2, bt, lens2)

    return out.reshape(B, 1, H, D)


# device time: 64744 ns/iter; 1.1417x vs baseline; 1.1134x over previous
import jax
import jax.numpy as jnp
from jax import lax
from jax.experimental import pallas as pl
from jax.experimental.pallas import tpu as pltpu


def kernel(Q, K, V, bt, lens):
    B, _, H, D = Q.shape
    P_loc, BS = K.shape[0], K.shape[1]
    NB = bt.shape[1]
    KT = P_loc * BS
    HD = H * D
    Hh = H // 2
    HDh = Hh * D
    HBh = Hh * B
    scale = D ** -0.5

    mx = lax.axis_index("x")

    Qt = Q.reshape(B, H, D).transpose(1, 0, 2) * scale
    eye = jnp.eye(Hh, dtype=Q.dtype)
    Qb0 = jnp.einsum("hbd,hg->hbgd", Qt[:Hh], eye).reshape(HBh, HDh)
    Qb1 = jnp.einsum("hbd,hg->hbgd", Qt[Hh:], eye).reshape(HBh, HDh)
    Qbig = jnp.where(mx == 0, Qb0, Qb1)

    K3 = K.reshape(KT, H, D)
    V3 = V.reshape(KT, H, D)
    lens2 = lens.reshape(B, 1)

    def body(q_ref, k_ref, v_ref, bt_ref, lens_ref, out_ref,
             k2, v2, sendy, recvy, sendx, recvx, sems_s, sems_r):
        my_x = lax.axis_index("x")
        my_y = lax.axis_index("y")
        ypeer = (my_x, 1 - my_y)
        xpeer = (1 - my_x, my_y)

        barrier = pltpu.get_barrier_semaphore()
        for nbr in (ypeer, xpeer):
            pl.semaphore_signal(barrier, inc=1, device_id=nbr,
                                device_id_type=pl.DeviceIdType.MESH)
        pl.semaphore_wait(barrier, 2)

        @pl.when(my_x == 0)
        def _():
            for hl in range(Hh):
                sl = pl.ds(hl * D, D)
                k2[:, sl] = k_ref[:, hl, :]
                v2[:, sl] = v_ref[:, hl, :]

        @pl.when(my_x == 1)
        def _():
            for hl in range(Hh):
                sl = pl.ds(hl * D, D)
                k2[:, sl] = k_ref[:, Hh + hl, :]
                v2[:, sl] = v_ref[:, Hh + hl, :]

        valid = lax.broadcasted_iota(jnp.int32, (B, NB), 1) < lens_ref[:, :]
        pg = my_y * P_loc + lax.broadcasted_iota(jnp.int32, (P_loc, B, NB), 0)
        hit = (bt_ref[:, :][None, :, :] == pg) & valid[None, :, :]
        counts = jnp.sum(jnp.where(hit, 1.0, 0.0), axis=2)

        kp = lax.broadcasted_iota(jnp.int32, (P_loc, KT), 1) // BS
        pp = lax.broadcasted_iota(jnp.int32, (P_loc, KT), 0)
        E = jnp.where(kp == pp, 1.0, 0.0)
        w = lax.dot_general(counts, E, (((0,), (0,)), ((), ())),
                            preferred_element_type=jnp.float32)
        rb = lax.broadcasted_iota(jnp.int32, (B, HBh), 1) % B
        bb = lax.broadcasted_iota(jnp.int32, (B, HBh), 0)
        T = jnp.where(rb == bb, 1.0, 0.0)
        wbig = lax.dot_general(T, w, (((0,), (0,)), ((), ())),
                               preferred_element_type=jnp.float32)

        s = lax.dot_general(q_ref[:, :], k2[:, :],
                            (((1,), (1,)), ((), ())),
                            preferred_element_type=jnp.float32)
        p = jnp.exp(s) * wbig

        obig = lax.dot_general(p, v2[:, :], (((1,), (0,)), ((), ())),
                               preferred_element_type=jnp.float32)
        lbig = lax.dot_general(p, jnp.ones((KT, D), jnp.float32),
                               (((1,), (0,)), ((), ())),
                               preferred_element_type=jnp.float32)

        for hl in range(Hh):
            sl = pl.ds(hl * D, D)
            sendy[:, sl] = obig[hl * B:(hl + 1) * B, hl * D:(hl + 1) * D]
            sendy[:, pl.ds(HDh + hl * D, D)] = lbig[hl * B:(hl + 1) * B, :]

        rdma_y = pltpu.make_async_remote_copy(
            src_ref=sendy, dst_ref=recvy,
            send_sem=sems_s.at[0], recv_sem=sems_r.at[0],
            device_id=ypeer, device_id_type=pl.DeviceIdType.MESH)
        rdma_y.start()
        rdma_y.wait()

        o_tot = sendy[:, :HDh] + recvy[:, :HDh]
        l_tot = sendy[:, HDh:] + recvy[:, HDh:]
        sendx[:, :] = o_tot / l_tot

        rdma_x = pltpu.make_async_remote_copy(
            src_ref=sendx, dst_ref=recvx,
            send_sem=sems_s.at[1], recv_sem=sems_r.at[1],
            device_id=xpeer, device_id_type=pl.DeviceIdType.MESH)
        rdma_x.start()
        rdma_x.wait()

        @pl.when(my_x == 0)
        def _():
            out_ref[:, :HDh] = sendx[:, :]
            out_ref[:, HDh:] = recvx[:, :]

        @pl.when(my_x == 1)
        def _():
            out_ref[:, :HDh] = recvx[:, :]
            out_ref[:, HDh:] = sendx[:, :]

    out = pl.pallas_call(
        body,
        out_shape=jax.ShapeDtypeStruct((B, HD), jnp.float32),
        in_specs=[pl.BlockSpec(memory_space=pltpu.VMEM)] * 5,
        out_specs=pl.BlockSpec(memory_space=pltpu.VMEM),
        scratch_shapes=[
            pltpu.VMEM((KT, HDh), jnp.float32),
            pltpu.VMEM((KT, HDh), jnp.float32),
            pltpu.VMEM((B, 2 * HDh), jnp.float32),
            pltpu.VMEM((B, 2 * HDh), jnp.float32),
            pltpu.VMEM((B, HDh), jnp.float32),
            pltpu.VMEM((B, HDh), jnp.float32),
            pltpu.SemaphoreType.DMA((2,)),
            pltpu.SemaphoreType.DMA((2,)),
        ],
        compiler_params=pltpu.CompilerParams(
            collective_id=0, vmem_limit_bytes=100 * 1024 * 1024),
    )(Qbig, K3, V3, bt, lens2)

    return out.reshape(B, 1, H, D)


# device time: 22694 ns/iter; 3.2571x vs baseline; 2.8529x over previous
import jax
import jax.numpy as jnp
from jax import lax
from jax.experimental import pallas as pl
from jax.experimental.pallas import tpu as pltpu


def kernel(Q, K, V, bt, lens):
    B, _, H, D = Q.shape
    P_loc, BS = K.shape[0], K.shape[1]
    NB = bt.shape[1]
    KT = P_loc * BS
    HD = H * D
    HB = H * B
    scale = D ** -0.5

    Qt = Q.reshape(B, H, D).transpose(1, 0, 2) * scale
    Qbig = jnp.einsum("hbd,hg->hbgd", Qt, jnp.eye(H, dtype=Q.dtype))
    Qbig = Qbig.reshape(HB, HD)

    KT2 = K.transpose(1, 2, 3, 0).reshape(BS * HD, P_loc)
    VT2 = V.transpose(1, 2, 3, 0).reshape(BS * HD, P_loc)
    lens2 = lens.reshape(B, 1)

    def body(q_ref, k_ref, v_ref, bt_ref, lens_ref, out_ref,
             send_buf, recv_buf, send_sem, recv_sem):
        my_x = lax.axis_index("x")
        my_y = lax.axis_index("y")
        peer = (my_x, 1 - my_y)

        barrier = pltpu.get_barrier_semaphore()
        pl.semaphore_signal(barrier, inc=1, device_id=peer,
                            device_id_type=pl.DeviceIdType.MESH)
        pl.semaphore_wait(barrier, 1)

        valid = lax.broadcasted_iota(jnp.int32, (B, NB), 1) < lens_ref[:, :]
        pg = my_y * P_loc + lax.broadcasted_iota(jnp.int32, (P_loc, B, NB), 0)
        hit = (bt_ref[:, :][None, :, :] == pg) & valid[None, :, :]
        counts = jnp.sum(jnp.where(hit, 1.0, 0.0), axis=2)

        kp = lax.broadcasted_iota(jnp.int32, (P_loc, KT), 1) % P_loc
        pp = lax.broadcasted_iota(jnp.int32, (P_loc, KT), 0)
        E = jnp.where(kp == pp, 1.0, 0.0)
        w = lax.dot_general(counts, E, (((0,), (0,)), ((), ())),
                            preferred_element_type=jnp.float32)
        rb = lax.broadcasted_iota(jnp.int32, (B, HB), 1) % B
        bb = lax.broadcasted_iota(jnp.int32, (B, HB), 0)
        T = jnp.where(rb == bb, 1.0, 0.0)
        wbig = lax.dot_general(T, w, (((0,), (0,)), ((), ())),
                               preferred_element_type=jnp.float32)

        q = q_ref[:, :]
        s_cols = []
        for t in range(BS):
            k_t = k_ref[t * HD:(t + 1) * HD, :]
            s_cols.append(lax.dot_general(
                q, k_t, (((1,), (0,)), ((), ())),
                preferred_element_type=jnp.float32))
        s = jnp.concatenate(s_cols, axis=1)
        p = jnp.exp(s) * wbig

        obig = None
        for t in range(BS):
            v_t = v_ref[t * HD:(t + 1) * HD, :]
            p_t = p[:, t * P_loc:(t + 1) * P_loc]
            o_t = lax.dot_general(p_t, v_t, (((1,), (1,)), ((), ())),
                                  preferred_element_type=jnp.float32)
            obig = o_t if obig is None else obig + o_t
        lbig = lax.dot_general(p, jnp.ones((KT, D), jnp.float32),
                               (((1,), (0,)), ((), ())),
                               preferred_element_type=jnp.float32)

        for h in range(H):
            sl = pl.ds(h * D, D)
            send_buf[:, sl] = obig[h * B:(h + 1) * B, h * D:(h + 1) * D]
            send_buf[:, pl.ds(HD + h * D, D)] = lbig[h * B:(h + 1) * B, :]

        rdma = pltpu.make_async_remote_copy(
            src_ref=send_buf, dst_ref=recv_buf,
            send_sem=send_sem, recv_sem=recv_sem,
            device_id=peer, device_id_type=pl.DeviceIdType.MESH)
        rdma.start()
        rdma.wait()

        o_tot = send_buf[:, :HD] + recv_buf[:, :HD]
        l_tot = send_buf[:, HD:] + recv_buf[:, HD:]
        out_ref[:, :] = o_tot / l_tot

    out = pl.pallas_call(
        body,
        out_shape=jax.ShapeDtypeStruct((B, HD), jnp.float32),
        in_specs=[pl.BlockSpec(memory_space=pltpu.VMEM)] * 5,
        out_specs=pl.BlockSpec(memory_space=pltpu.VMEM),
        scratch_shapes=[
            pltpu.VMEM((B, 2 * HD), jnp.float32),
            pltpu.VMEM((B, 2 * HD), jnp.float32),
            pltpu.SemaphoreType.DMA,
            pltpu.SemaphoreType.DMA,
        ],
        compiler_params=pltpu.CompilerParams(
            collective_id=0, vmem_limit_bytes=100 * 1024 * 1024),
    )(Qbig, KT2, VT2, bt, lens2)

    return out.reshape(B, 1, H, D)
